# Initial kernel scaffold; baseline (speedup 1.0000x reference)
import jax
import jax.numpy as jnp
from jax.experimental import pallas as pl


def kernel(user_hidden_out, item_hidden_out, fake_user_hidden_out, fake_item_hidden_out, UV_adj, VU_adj, CUV_adj, CVU_adj, user_One, item_One, UV_rated, UV_rated_vals, VU_rated, VU_rated_vals, relation_UV_adj, lin1_W, lin1_b, lin2_W, lin2_b, lin_W, lin_b, lin_sub_W, lin_sub_b, fk_W, fk_b, gat_Wu, gat_Wv):
    raise NotImplementedError("write your pallas kernel here")


# speedup vs baseline: 10.2354x; 10.2354x over previous
import functools

import jax
import jax.numpy as jnp
from jax import lax
from jax.experimental import pallas as pl
from jax.experimental.pallas import tpu as pltpu
from jax.experimental.pallas import tpu_sc as plsc

N = 10000
D = 128
E = 320000
B = 4096
NC, NS = 2, 16
NW = NC * NS
CH = 80
K = 8
ER = E // CH
NSUPER = ER // K
NPAD = 10240
BPT = B // NW

_MESH = plsc.VectorSubcoreMesh(core_axis_name="c", subcore_axis_name="s",
                               num_cores=NC, num_subcores=NS)
_SC_PARAMS = pltpu.CompilerParams(needs_layout_passes=False)
_f32 = jnp.float32
_i32 = jnp.int32


def _fill16(ref, n, val):
    v = jnp.full((16,), val, _f32)
    for m in range(n // 16):
        ref[pl.ds(16 * m, 16)] = v


def _sc_agg_body(tab_u, tab_i, tab_fu, tab_fi,
                 d_u, s_u, d_i, s_i, d_fu, s_fu, d_fi, s_fi,
                 zrows, z640,
                 agg_u0, agg_u1, agg_i0, agg_i1,
                 agg_fu0, agg_fu1, agg_fi0, agg_fi1,
                 deg_u0, deg_u1, deg_i0, deg_i1,
                 deg_fu0, deg_fu1, deg_fi0, deg_fi1,
                 acc_sh, deg_sh, sidx, didx, rbuf, ones_v,
                 sem_i, sem_g, sem_s):
    c = lax.axis_index("c")
    s = lax.axis_index("s")
    w = c * NS + s
    nsup = jnp.where(w < NSUPER - 15 * NW, 16, 15)
    _fill16(ones_v, CH, 1.0)

    branches = (
        (tab_i, d_u, s_u, agg_u0, agg_u1, deg_u0, deg_u1),
        (tab_u, d_i, s_i, agg_i0, agg_i1, deg_i0, deg_i1),
        (tab_fi, d_fu, s_fu, agg_fu0, agg_fu1, deg_fu0, deg_fu1),
        (tab_fu, d_fi, s_fi, agg_fi0, agg_fi1, deg_fi0, deg_fi1),
    )
    for tab, d2d, s2d, aggO0, aggO1, degO0, degO1 in branches:
        pltpu.sync_copy(zrows, acc_sh.at[pl.ds(640 * s, 640)])
        pltpu.sync_copy(z640, deg_sh.at[pl.ds(640 * s, 640)])
        plsc.subcore_barrier()

        def body(j, carry):
            r0 = 8 * (w + NW * j)
            cp_s = pltpu.async_copy(s2d.at[pl.ds(r0, K)], sidx, sem_i)
            cp_d = pltpu.async_copy(d2d.at[pl.ds(r0, K)], didx, sem_i)
            cp_s.wait()
            cp_d.wait()
            sg = [None] * K
            sa = [None] * K
            sd = [None] * K
            for b in range(K):
                if b >= 2:
                    sa[b - 2].wait()
                    sd[b - 2].wait()
                sg[b] = pltpu.async_copy(tab.at[sidx.at[b]],
                                         rbuf.at[b % 2], sem_g)
                if b >= 1:
                    sg[b - 1].wait()
                    sa[b - 1] = pltpu.async_copy(
                        rbuf.at[(b - 1) % 2], acc_sh.at[didx.at[b - 1]],
                        sem_s, add=True)
                    sd[b - 1] = pltpu.async_copy(
                        ones_v, deg_sh.at[didx.at[b - 1]], sem_s, add=True)
            sg[K - 1].wait()
            sa[K - 1] = pltpu.async_copy(
                rbuf.at[(K - 1) % 2], acc_sh.at[didx.at[K - 1]],
                sem_s, add=True)
            sd[K - 1] = pltpu.async_copy(
                ones_v, deg_sh.at[didx.at[K - 1]], sem_s, add=True)
            for b in (K - 2, K - 1):
                sa[b].wait()
                sd[b].wait()
            return carry

        lax.fori_loop(0, nsup, body, 0)
        plsc.subcore_barrier()

        @pl.when(c == 0)
        def _():
            pltpu.sync_copy(acc_sh.at[pl.ds(640 * s, 640)],
                            aggO0.at[pl.ds(640 * s, 640)])
            pltpu.sync_copy(deg_sh.at[pl.ds(640 * s, 640)],
                            degO0.at[pl.ds(640 * s, 640)])

        @pl.when(c == 1)
        def _():
            pltpu.sync_copy(acc_sh.at[pl.ds(640 * s, 640)],
                            aggO1.at[pl.ds(640 * s, 640)])
            pltpu.sync_copy(deg_sh.at[pl.ds(640 * s, 640)],
                            degO1.at[pl.ds(640 * s, 640)])


_sc_agg = functools.partial(
    pl.kernel,
    out_type=[jax.ShapeDtypeStruct((NPAD, D), _f32)] * 8
             + [jax.ShapeDtypeStruct((NPAD,), _f32)] * 8,
    mesh=_MESH,
    compiler_params=_SC_PARAMS,
    scratch_types=[
        pltpu.VMEM_SHARED((NPAD, D), _f32),
        pltpu.VMEM_SHARED((NPAD,), _f32),
        pltpu.VMEM((K, CH), _i32),
        pltpu.VMEM((K, CH), _i32),
        pltpu.VMEM((2, CH, D), _f32),
        pltpu.VMEM((CH,), _f32),
        pltpu.SemaphoreType.DMA,
        pltpu.SemaphoreType.DMA,
        pltpu.SemaphoreType.DMA,
    ],
)(_sc_agg_body)


def _sc_cw_body(r_uv, c_uv, v_uv, r_vu, c_vu, v_vu, z640,
                cw_uv0, cw_uv1, cw_vu0, cw_vu1,
                den_uv_sh, den_vu_sh, cwuv_sh, cwvu_sh,
                ridx, cidx, vbuf, wbuf, dloc,
                sem_i, sem_s):
    c = lax.axis_index("c")
    s = lax.axis_index("s")
    w = c * NS + s
    nsup1 = jnp.where(s < NSUPER - 31 * NS, 32, 31)
    nsup2 = jnp.where(w < NSUPER - 15 * NW, 16, 15)
    for sh in (den_uv_sh, den_vu_sh, cwuv_sh, cwvu_sh):
        pltpu.sync_copy(z640, sh.at[pl.ds(640 * s, 640)])
    plsc.subcore_barrier()

    for r2d, v2d, den_sh in ((r_uv, v_uv, den_uv_sh), (r_vu, v_vu, den_vu_sh)):
        def dbody(j, carry):
            r0 = 8 * (s + NS * j)
            cp_r = pltpu.async_copy(r2d.at[pl.ds(r0, K)], ridx, sem_i)
            cp_v = pltpu.async_copy(v2d.at[pl.ds(r0, K)], vbuf, sem_i)
            cp_r.wait()
            cp_v.wait()
            ss = [pltpu.async_copy(vbuf.at[b], den_sh.at[ridx.at[b]], sem_s,
                                   add=True) for b in range(K)]
            for x in ss:
                x.wait()
            return carry

        lax.fori_loop(0, nsup1, dbody, 0)
    plsc.subcore_barrier()

    for r2d, c2d, v2d, den_sh, cw_sh in (
            (r_uv, c_uv, v_uv, den_uv_sh, cwuv_sh),
            (r_vu, c_vu, v_vu, den_vu_sh, cwvu_sh)):
        pltpu.sync_copy(den_sh, dloc)

        def rbody(i, carry):
            d = dloc[pl.ds(16 * i, 16)]
            dloc[pl.ds(16 * i, 16)] = 1.0 / jnp.where(d > 0.0, d, 1.0)
            return carry

        lax.fori_loop(0, NPAD // 16, rbody, 0)

        def cbody(j, carry):
            r0 = 8 * (w + NW * j)
            cp_r = pltpu.async_copy(r2d.at[pl.ds(r0, K)], ridx, sem_i)
            cp_c = pltpu.async_copy(c2d.at[pl.ds(r0, K)], cidx, sem_i)
            cp_v = pltpu.async_copy(v2d.at[pl.ds(r0, K)], vbuf, sem_i)
            cp_r.wait()
            cp_c.wait()
            cp_v.wait()
            for b in range(K):
                for m in range(CH // 16):
                    ri = ridx[b, pl.ds(16 * m, 16)]
                    rc = plsc.load_gather(dloc, [ri])
                    wbuf[b, pl.ds(16 * m, 16)] = (
                        vbuf[b, pl.ds(16 * m, 16)] * rc)
            ss = [pltpu.async_copy(wbuf.at[b], cw_sh.at[cidx.at[b]], sem_s,
                                   add=True) for b in range(K)]
            for x in ss:
                x.wait()
            return carry

        lax.fori_loop(0, nsup2, cbody, 0)
    plsc.subcore_barrier()

    for cw_sh, cwO0, cwO1 in ((cwuv_sh, cw_uv0, cw_uv1),
                              (cwvu_sh, cw_vu0, cw_vu1)):
        @pl.when(c == 0)
        def _():
            pltpu.sync_copy(cw_sh.at[pl.ds(640 * s, 640)],
                            cwO0.at[pl.ds(640 * s, 640)])

        @pl.when(c == 1)
        def _():
            pltpu.sync_copy(cw_sh.at[pl.ds(640 * s, 640)],
                            cwO1.at[pl.ds(640 * s, 640)])


_sc_cw = functools.partial(
    pl.kernel,
    out_type=[jax.ShapeDtypeStruct((NPAD,), _f32)] * 4,
    mesh=_MESH,
    compiler_params=_SC_PARAMS,
    scratch_types=[
        pltpu.VMEM_SHARED((NPAD,), _f32),
        pltpu.VMEM_SHARED((NPAD,), _f32),
        pltpu.VMEM_SHARED((NPAD,), _f32),
        pltpu.VMEM_SHARED((NPAD,), _f32),
        pltpu.VMEM((K, CH), _i32),
        pltpu.VMEM((K, CH), _i32),
        pltpu.VMEM((K, CH), _f32),
        pltpu.VMEM((K, CH), _f32),
        pltpu.VMEM((NPAD,), _f32),
        pltpu.SemaphoreType.DMA,
        pltpu.SemaphoreType.DMA,
    ],
)(_sc_cw_body)


def _sc_gather_body(agg_u0, agg_u1, agg_i0, agg_i1,
                    agg_fu0, agg_fu1, agg_fi0, agg_fi1,
                    deg_u0, deg_u1, deg_i0, deg_i1,
                    deg_fu0, deg_fu1, deg_fi0, deg_fi1,
                    tab_u, tab_i, tab_fu, tab_fi,
                    one_u, one_i,
                    nu_u, t_u, nu_i, t_i, nu_fu, t_fu, nu_fi, t_fi,
                    idxu, idxi, r0b, r1b, tb, nub, d0loc, d1loc, rvb,
                    sem_g):
    c = lax.axis_index("c")
    s = lax.axis_index("s")
    base = (c * NS + s) * BPT
    pltpu.sync_copy(one_u.at[pl.ds(base, BPT)], idxu)
    pltpu.sync_copy(one_i.at[pl.ds(base, BPT)], idxi)

    branches = (
        (agg_u0, agg_u1, deg_u0, deg_u1, tab_u, idxu, nu_u, t_u),
        (agg_i0, agg_i1, deg_i0, deg_i1, tab_i, idxi, nu_i, t_i),
        (agg_fu0, agg_fu1, deg_fu0, deg_fu1, tab_fu, idxu, nu_fu, t_fu),
        (agg_fi0, agg_fi1, deg_fi0, deg_fi1, tab_fi, idxi, nu_fi, t_fi),
    )
    for a0, a1, dg0, dg1, tab, idx, nuO, tO in branches:
        cp0 = pltpu.async_copy(a0.at[idx], r0b, sem_g)
        cp1 = pltpu.async_copy(a1.at[idx], r1b, sem_g)
        cpt = pltpu.async_copy(tab.at[idx], tb, sem_g)
        pltpu.sync_copy(dg0, d0loc)
        pltpu.sync_copy(dg1, d1loc)
        cp0.wait()
        cp1.wait()
        cpt.wait()
        for m in range(BPT // 16):
            i16 = idx[pl.ds(16 * m, 16)]
            d0 = plsc.load_gather(d0loc, [i16])
            d1 = plsc.load_gather(d1loc, [i16])
            rvb[pl.ds(16 * m, 16)] = 1.0 / jnp.maximum(d0 + d1, 1.0)

        def nbody(k, carry):
            sc = plsc.load_gather(rvb, [jnp.full((16,), k, _i32)])
            for j in range(D // 16):
                nub[k, pl.ds(16 * j, 16)] = (
                    r0b[k, pl.ds(16 * j, 16)]
                    + r1b[k, pl.ds(16 * j, 16)]) * sc
            return carry

        lax.fori_loop(0, BPT, nbody, 0)
        pltpu.sync_copy(nub, nuO.at[pl.ds(base, BPT)])
        pltpu.sync_copy(tb, tO.at[pl.ds(base, BPT)])


_sc_gather = functools.partial(
    pl.kernel,
    out_type=[jax.ShapeDtypeStruct((B, D), _f32)] * 8,
    mesh=_MESH,
    compiler_params=_SC_PARAMS,
    scratch_types=[
        pltpu.VMEM((BPT,), _i32),
        pltpu.VMEM((BPT,), _i32),
        pltpu.VMEM((BPT, D), _f32),
        pltpu.VMEM((BPT, D), _f32),
        pltpu.VMEM((BPT, D), _f32),
        pltpu.VMEM((BPT, D), _f32),
        pltpu.VMEM((NPAD,), _f32),
        pltpu.VMEM((NPAD,), _f32),
        pltpu.VMEM((BPT,), _f32),
        pltpu.SemaphoreType.DMA,
    ],
)(_sc_gather_body)


def _tc_reduce_body(uh_ref, ih_ref, cwuv_ref, cwvu_ref, acc_ref):
    i = pl.program_id(0)

    @pl.when(i == 0)
    def _():
        acc_ref[...] = jnp.zeros_like(acc_ref)

    uhb = uh_ref[...]
    ihb = ih_ref[...]
    cu = cwuv_ref[:, 0:1] + cwuv_ref[:, 1:2]
    cv = cwvu_ref[:, 0:1] + cwvu_ref[:, 1:2]
    acc_ref[0:1, :] += jnp.sum(uhb, axis=0, keepdims=True)
    acc_ref[1:2, :] += jnp.sum(ihb, axis=0, keepdims=True)
    acc_ref[2:3, :] += jnp.sum(ihb * cu, axis=0, keepdims=True)
    acc_ref[3:4, :] += jnp.sum(uhb * cv, axis=0, keepdims=True)


def _tc_reduce(uh, ih, cw_uv, cw_vu):
    blk = 1000
    return pl.pallas_call(
        _tc_reduce_body,
        grid=(N // blk,),
        in_specs=[
            pl.BlockSpec((blk, D), lambda i: (i, 0)),
            pl.BlockSpec((blk, D), lambda i: (i, 0)),
            pl.BlockSpec((blk, NC), lambda i: (i, 0)),
            pl.BlockSpec((blk, NC), lambda i: (i, 0)),
        ],
        out_specs=pl.BlockSpec((8, D), lambda i: (0, 0)),
        out_shape=jax.ShapeDtypeStruct((8, D), _f32),
    )(uh, ih, cw_uv, cw_vu)


def _tc_final_body(nu_u, t_u, nu_i, t_i, nu_fu, t_fu, nu_fi, t_fi, accr,
                   Wu, Wv, l1W, l1b, l2W, l2b, lW, lb, lsW, lsb, fkW, fkb,
                   pr_r, pr_f, lab_r, lab_f):
    inv_n = 1.0 / N
    acc = accr[...]
    S_u = acc[0:1, :] * inv_n
    S_i = acc[1:2, :] * inv_n
    G_icu = acc[2:3, :] * inv_n
    G_uci = acc[3:4, :] * inv_n
    g = jnp.dot(jnp.concatenate([S_u, G_icu], axis=1), l1W[...],
                preferred_element_type=_f32) + l1b[...]
    h = jnp.dot(jnp.concatenate([S_i, G_uci], axis=1), l2W[...],
                preferred_element_type=_f32) + l2b[...]
    st = jax.nn.sigmoid(jnp.dot((g + h) * 0.5, lW[...],
                                preferred_element_type=_f32) + lb[...])
    v = lax.dot_general(st, fkW[...], (((1,), (1,)), ((), ())),
                        preferred_element_type=_f32)

    ru = jnp.maximum(jnp.dot(nu_u[...], Wu[...],
                             preferred_element_type=_f32) + t_u[...], 0.0)
    ri = jnp.maximum(jnp.dot(nu_i[...], Wv[...],
                             preferred_element_type=_f32) + t_i[...], 0.0)
    fu = jnp.maximum(jnp.dot(nu_fu[...], Wu[...],
                             preferred_element_type=_f32) + t_fu[...], 0.0)
    fi = jnp.maximum(jnp.dot(nu_fi[...], Wv[...],
                             preferred_element_type=_f32) + t_fi[...], 0.0)
    lsW1 = lsW[0:D, :]
    lsW2 = lsW[D:2 * D, :]
    rs = jax.nn.sigmoid(jnp.dot(ru, lsW1, preferred_element_type=_f32)
                        + jnp.dot(ri, lsW2, preferred_element_type=_f32)
                        + lsb[...])
    fs = jax.nn.sigmoid(jnp.dot(fu, lsW1, preferred_element_type=_f32)
                        + jnp.dot(fi, lsW2, preferred_element_type=_f32)
                        + lsb[...])
    bias = fkb[...]
    pr_r[...] = jax.nn.sigmoid(
        jnp.sum(rs * v, axis=1, keepdims=True) + bias)
    pr_f[...] = jax.nn.sigmoid(
        jnp.sum(fs * v, axis=1, keepdims=True) + bias)
    lab_r[...] = jnp.ones_like(lab_r)
    lab_f[...] = jnp.zeros_like(lab_f)


def _tc_final(nu_u, t_u, nu_i, t_i, nu_fu, t_fu, nu_fi, t_fi, acc,
              Wu, Wv, l1W, l1b, l2W, l2b, lW, lb, lsW, lsb, fkW, fkb):
    blk = 512
    row = lambda i: (i, 0)
    fixed = lambda i: (0, 0)
    bspec = pl.BlockSpec((blk, D), row)
    return pl.pallas_call(
        _tc_final_body,
        grid=(B // blk,),
        in_specs=[bspec] * 8 + [
            pl.BlockSpec((8, D), fixed),
            pl.BlockSpec((D, D), fixed),
            pl.BlockSpec((D, D), fixed),
            pl.BlockSpec((2 * D, D), fixed),
            pl.BlockSpec((1, D), fixed),
            pl.BlockSpec((2 * D, D), fixed),
            pl.BlockSpec((1, D), fixed),
            pl.BlockSpec((D, D), fixed),
            pl.BlockSpec((1, D), fixed),
            pl.BlockSpec((2 * D, D), fixed),
            pl.BlockSpec((1, D), fixed),
            pl.BlockSpec((D, D), fixed),
            pl.BlockSpec((1, 1), fixed),
        ],
        out_specs=[pl.BlockSpec((blk, 1), row)] * 4,
        out_shape=[jax.ShapeDtypeStruct((B, 1), _f32)] * 4,
    )(nu_u, t_u, nu_i, t_i, nu_fu, t_fu, nu_fi, t_fi, acc,
      Wu, Wv, l1W, l1b, l2W, l2b, lW, lb, lsW, lsb, fkW, fkb)


def _2d(a):
    return a.reshape(ER, CH)


def kernel(user_hidden_out, item_hidden_out, fake_user_hidden_out,
           fake_item_hidden_out, UV_adj, VU_adj, CUV_adj, CVU_adj,
           user_One, item_One, UV_rated, UV_rated_vals, VU_rated,
           VU_rated_vals, relation_UV_adj, lin1_W, lin1_b, lin2_W, lin2_b,
           lin_W, lin_b, lin_sub_W, lin_sub_b, fk_W, fk_b, gat_Wu, gat_Wv):
    del relation_UV_adj
    zrows = jnp.zeros((640, D), _f32)
    z640 = jnp.zeros((640,), _f32)

    (agg_u0, agg_u1, agg_i0, agg_i1, agg_fu0, agg_fu1, agg_fi0, agg_fi1,
     deg_u0, deg_u1, deg_i0, deg_i1, deg_fu0, deg_fu1, deg_fi0,
     deg_fi1) = _sc_agg(
        user_hidden_out, item_hidden_out, fake_user_hidden_out,
        fake_item_hidden_out,
        _2d(UV_adj[0]), _2d(UV_adj[1]), _2d(VU_adj[0]), _2d(VU_adj[1]),
        _2d(CUV_adj[0]), _2d(CUV_adj[1]), _2d(CVU_adj[0]), _2d(CVU_adj[1]),
        zrows, z640)

    cw_uv0, cw_uv1, cw_vu0, cw_vu1 = _sc_cw(
        _2d(UV_rated[0]), _2d(UV_rated[1]), _2d(UV_rated_vals),
        _2d(VU_rated[0]), _2d(VU_rated[1]), _2d(VU_rated_vals), z640)

    (nu_u, t_u, nu_i, t_i, nu_fu, t_fu, nu_fi, t_fi) = _sc_gather(
        agg_u0, agg_u1, agg_i0, agg_i1, agg_fu0, agg_fu1, agg_fi0, agg_fi1,
        deg_u0, deg_u1, deg_i0, deg_i1, deg_fu0, deg_fu1, deg_fi0, deg_fi1,
        user_hidden_out, item_hidden_out, fake_user_hidden_out,
        fake_item_hidden_out, user_One, item_One)

    acc = _tc_reduce(user_hidden_out, item_hidden_out,
                     jnp.stack([cw_uv0[:N], cw_uv1[:N]], axis=1),
                     jnp.stack([cw_vu0[:N], cw_vu1[:N]], axis=1))

    pr_r, pr_f, lab_r, lab_f = _tc_final(
        nu_u, t_u, nu_i, t_i, nu_fu, t_fu, nu_fi, t_fi, acc,
        gat_Wu, gat_Wv, lin1_W, lin1_b.reshape(1, D), lin2_W,
        lin2_b.reshape(1, D), lin_W, lin_b.reshape(1, D), lin_sub_W,
        lin_sub_b.reshape(1, D), fk_W, fk_b.reshape(1, 1))

    prob = jnp.concatenate([pr_r.reshape(B), pr_f.reshape(B)])
    label = jnp.concatenate([lab_r.reshape(B), lab_f.reshape(B)])
    return prob, label


# speedup vs baseline: 15.2686x; 1.4917x over previous
import functools

import jax
import jax.numpy as jnp
from jax import lax
from jax.experimental import pallas as pl
from jax.experimental.pallas import tpu as pltpu
from jax.experimental.pallas import tpu_sc as plsc

N = 10000
D = 128
E = 320000
B = 4096
NC, NS = 2, 16
NW = NC * NS
CH = 80
K = 8
ER = E // CH
NSUPER = ER // K
NPAD = 10240
BPT = B // NW

_MESH = plsc.VectorSubcoreMesh(core_axis_name="c", subcore_axis_name="s",
                               num_cores=NC, num_subcores=NS)
_SC_PARAMS = pltpu.CompilerParams(needs_layout_passes=False)
_f32 = jnp.float32
_i32 = jnp.int32


def _fill16(ref, n, val):
    v = jnp.full((16,), val, _f32)
    for m in range(n // 16):
        ref[pl.ds(16 * m, 16)] = v


def _sc_agg_body(tab_u, tab_i, tab_fu, tab_fi,
                 uv3, vu3, cuv3, cvu3,
                 zrows, z640,
                 agg_u, agg_i, agg_fu, agg_fi,
                 deg_u, deg_i, deg_fu, deg_fi,
                 acc_sh, deg_sh, sidx, didx, rbuf, ones_v,
                 sem_i, sem_g, sem_s):
    c = lax.axis_index("c")
    s = lax.axis_index("s")
    nsup = jnp.where(s < NSUPER - 31 * NS, 32, 31)
    _fill16(ones_v, CH, 1.0)

    def accumulate(tab, e3):
        d3 = e3.at[0]
        s3 = e3.at[1]
        pltpu.async_copy(s3.at[pl.ds(8 * s, K)], sidx.at[0], sem_i)
        pltpu.async_copy(d3.at[pl.ds(8 * s, K)], didx.at[0], sem_i)

        def body(j, carry):
            q = jnp.bitwise_and(j, 1)
            pltpu.make_async_copy(s3.at[pl.ds(0, K)], sidx.at[0],
                                  sem_i).wait()
            pltpu.make_async_copy(d3.at[pl.ds(0, K)], didx.at[0],
                                  sem_i).wait()

            @pl.when(j < nsup - 1)
            def _():
                r0n = 8 * (s + NS * (j + 1))
                pltpu.async_copy(s3.at[pl.ds(r0n, K)], sidx.at[1 - q],
                                 sem_i)
                pltpu.async_copy(d3.at[pl.ds(r0n, K)], didx.at[1 - q],
                                 sem_i)

            sg = [None] * K
            sa = [None] * K
            sd = [None] * K

            def scat(b):
                sg[b].wait()
                sa[b] = pltpu.async_copy(
                    rbuf.at[b % 4], acc_sh.at[didx.at[q, b]],
                    sem_s, add=True)
                sd[b] = pltpu.async_copy(
                    ones_v, deg_sh.at[didx.at[q, b]], sem_s, add=True)

            for b in range(K):
                if b >= 4:
                    sa[b - 4].wait()
                    sd[b - 4].wait()
                sg[b] = pltpu.async_copy(tab.at[sidx.at[q, b]],
                                         rbuf.at[b % 4], sem_g)
                if b >= 2:
                    scat(b - 2)
            scat(K - 2)
            scat(K - 1)
            for b in range(K - 4, K):
                sa[b].wait()
                sd[b].wait()
            return carry

        lax.fori_loop(0, nsup, body, 0)

    def copyout(aggO, degO):
        pltpu.sync_copy(acc_sh.at[pl.ds(640 * s, 640)],
                        aggO.at[pl.ds(640 * s, 640)])
        pltpu.sync_copy(deg_sh.at[pl.ds(640 * s, 640)],
                        degO.at[pl.ds(640 * s, 640)])

    phases = (
        ((tab_i, uv3, agg_u, deg_u),
         (tab_fi, cuv3, agg_fu, deg_fu)),
        ((tab_u, vu3, agg_i, deg_i),
         (tab_fu, cvu3, agg_fi, deg_fi)),
    )
    for br0, br1 in phases:
        pltpu.sync_copy(zrows, acc_sh.at[pl.ds(640 * s, 640)])
        pltpu.sync_copy(z640, deg_sh.at[pl.ds(640 * s, 640)])
        plsc.subcore_barrier()

        @pl.when(c == 0)
        def _():
            accumulate(br0[0], br0[1])

        @pl.when(c == 1)
        def _():
            accumulate(br1[0], br1[1])

        plsc.subcore_barrier()

        @pl.when(c == 0)
        def _():
            copyout(br0[2], br0[3])

        @pl.when(c == 1)
        def _():
            copyout(br1[2], br1[3])


_sc_agg = functools.partial(
    pl.kernel,
    out_type=[jax.ShapeDtypeStruct((NPAD, D), _f32)] * 4
             + [jax.ShapeDtypeStruct((NPAD,), _f32)] * 4,
    mesh=_MESH,
    compiler_params=_SC_PARAMS,
    scratch_types=[
        pltpu.VMEM_SHARED((NPAD, D), _f32),
        pltpu.VMEM_SHARED((NPAD,), _f32),
        pltpu.VMEM((2, K, CH), _i32),
        pltpu.VMEM((2, K, CH), _i32),
        pltpu.VMEM((4, CH, D), _f32),
        pltpu.VMEM((CH,), _f32),
        pltpu.SemaphoreType.DMA,
        pltpu.SemaphoreType.DMA,
        pltpu.SemaphoreType.DMA,
    ],
)(_sc_agg_body)


def _sc_cw_body(uvr3, v_uv, vur3, v_vu, z640,
                cw_uv, cw_vu,
                den_sh, cw_sh,
                ridx, cidx, vbuf, wbuf, dloc,
                sem_i, sem_s):
    c = lax.axis_index("c")
    s = lax.axis_index("s")
    nsup = jnp.where(s < NSUPER - 31 * NS, 32, 31)
    pltpu.sync_copy(z640, den_sh.at[pl.ds(640 * s, 640)])
    pltpu.sync_copy(z640, cw_sh.at[pl.ds(640 * s, 640)])
    plsc.subcore_barrier()

    def den_pass(r2d, v2d):
        pltpu.async_copy(r2d.at[pl.ds(8 * s, K)], ridx.at[0], sem_i)
        pltpu.async_copy(v2d.at[pl.ds(8 * s, K)], vbuf.at[0], sem_i)

        def dbody(j, carry):
            q = jnp.bitwise_and(j, 1)
            pltpu.make_async_copy(r2d.at[pl.ds(0, K)], ridx.at[0],
                                  sem_i).wait()
            pltpu.make_async_copy(v2d.at[pl.ds(0, K)], vbuf.at[0],
                                  sem_i).wait()

            @pl.when(j < nsup - 1)
            def _():
                r0n = 8 * (s + NS * (j + 1))
                pltpu.async_copy(r2d.at[pl.ds(r0n, K)], ridx.at[1 - q],
                                 sem_i)
                pltpu.async_copy(v2d.at[pl.ds(r0n, K)], vbuf.at[1 - q],
                                 sem_i)

            ss = [pltpu.async_copy(vbuf.at[q, b], den_sh.at[ridx.at[q, b]],
                                   sem_s, add=True) for b in range(K)]
            for x in ss:
                x.wait()
            return carry

        lax.fori_loop(0, nsup, dbody, 0)

    def cw_pass(r2d, c2d, v2d):
        pltpu.sync_copy(den_sh, dloc)

        def rbody(i, carry):
            d = dloc[pl.ds(16 * i, 16)]
            dloc[pl.ds(16 * i, 16)] = 1.0 / jnp.where(d > 0.0, d, 1.0)
            return carry

        lax.fori_loop(0, NPAD // 16, rbody, 0)

        pltpu.async_copy(r2d.at[pl.ds(8 * s, K)], ridx.at[0], sem_i)
        pltpu.async_copy(c2d.at[pl.ds(8 * s, K)], cidx.at[0], sem_i)
        pltpu.async_copy(v2d.at[pl.ds(8 * s, K)], vbuf.at[0], sem_i)

        def cbody(j, carry):
            q = jnp.bitwise_and(j, 1)
            pltpu.make_async_copy(r2d.at[pl.ds(0, K)], ridx.at[0],
                                  sem_i).wait()
            pltpu.make_async_copy(c2d.at[pl.ds(0, K)], cidx.at[0],
                                  sem_i).wait()
            pltpu.make_async_copy(v2d.at[pl.ds(0, K)], vbuf.at[0],
                                  sem_i).wait()

            @pl.when(j < nsup - 1)
            def _():
                r0n = 8 * (s + NS * (j + 1))
                pltpu.async_copy(r2d.at[pl.ds(r0n, K)], ridx.at[1 - q],
                                 sem_i)
                pltpu.async_copy(c2d.at[pl.ds(r0n, K)], cidx.at[1 - q],
                                 sem_i)
                pltpu.async_copy(v2d.at[pl.ds(r0n, K)], vbuf.at[1 - q],
                                 sem_i)

            for b in range(K):
                for m in range(CH // 16):
                    ri = ridx[q, b, pl.ds(16 * m, 16)]
                    rc = plsc.load_gather(dloc, [ri])
                    wbuf[b, pl.ds(16 * m, 16)] = (
                        vbuf[q, b, pl.ds(16 * m, 16)] * rc)
            ss = [pltpu.async_copy(wbuf.at[b], cw_sh.at[cidx.at[q, b]],
                                   sem_s, add=True) for b in range(K)]
            for x in ss:
                x.wait()
            return carry

        lax.fori_loop(0, nsup, cbody, 0)

    @pl.when(c == 0)
    def _():
        den_pass(uvr3.at[0], v_uv)

    @pl.when(c == 1)
    def _():
        den_pass(vur3.at[0], v_vu)

    plsc.subcore_barrier()

    @pl.when(c == 0)
    def _():
        cw_pass(uvr3.at[0], uvr3.at[1], v_uv)

    @pl.when(c == 1)
    def _():
        cw_pass(vur3.at[0], vur3.at[1], v_vu)

    plsc.subcore_barrier()

    @pl.when(c == 0)
    def _():
        pltpu.sync_copy(cw_sh.at[pl.ds(640 * s, 640)],
                        cw_uv.at[pl.ds(640 * s, 640)])

    @pl.when(c == 1)
    def _():
        pltpu.sync_copy(cw_sh.at[pl.ds(640 * s, 640)],
                        cw_vu.at[pl.ds(640 * s, 640)])


_sc_cw = functools.partial(
    pl.kernel,
    out_type=[jax.ShapeDtypeStruct((NPAD,), _f32)] * 2,
    mesh=_MESH,
    compiler_params=_SC_PARAMS,
    scratch_types=[
        pltpu.VMEM_SHARED((NPAD,), _f32),
        pltpu.VMEM_SHARED((NPAD,), _f32),
        pltpu.VMEM((2, K, CH), _i32),
        pltpu.VMEM((2, K, CH), _i32),
        pltpu.VMEM((2, K, CH), _f32),
        pltpu.VMEM((K, CH), _f32),
        pltpu.VMEM((NPAD,), _f32),
        pltpu.SemaphoreType.DMA,
        pltpu.SemaphoreType.DMA,
    ],
)(_sc_cw_body)


def _sc_gather_body(agg_u, agg_i, agg_fu, agg_fi,
                    deg_u, deg_i, deg_fu, deg_fi,
                    tab_u, tab_i, tab_fu, tab_fi,
                    one_u, one_i,
                    ag_u, t_u, rv_u, ag_i, t_i, rv_i,
                    ag_fu, t_fu, rv_fu, ag_fi, t_fi, rv_fi,
                    idxu, idxi, r0a, tba, r0c, tbc, dega, degc,
                    rva, rvc, sem_g, sem_o):
    c = lax.axis_index("c")
    s = lax.axis_index("s")
    base = (c * NS + s) * BPT
    pltpu.sync_copy(one_u.at[pl.ds(base, BPT)], idxu)
    pltpu.sync_copy(one_i.at[pl.ds(base, BPT)], idxi)

    pairs = (
        ((agg_u, deg_u, tab_u, idxu, ag_u, t_u, rv_u),
         (agg_i, deg_i, tab_i, idxi, ag_i, t_i, rv_i)),
        ((agg_fu, deg_fu, tab_fu, idxu, ag_fu, t_fu, rv_fu),
         (agg_fi, deg_fi, tab_fi, idxi, ag_fi, t_fi, rv_fi)),
    )
    for pi, (brA, brB) in enumerate(pairs):
        aA, dA, tA, ixA, agOA, tOA, rvOA = brA
        aB, dB, tB, ixB, agOB, tOB, rvOB = brB
        cpA0 = pltpu.async_copy(aA.at[ixA], r0a, sem_g)
        cpAt = pltpu.async_copy(tA.at[ixA], tba, sem_g)
        cpB0 = pltpu.async_copy(aB.at[ixB], r0c, sem_g)
        cpBt = pltpu.async_copy(tB.at[ixB], tbc, sem_g)
        pltpu.sync_copy(dA, dega)
        for m in range(BPT // 16):
            d0 = plsc.load_gather(dega, [ixA[pl.ds(16 * m, 16)]])
            rva[pl.ds(16 * m, 16)] = 1.0 / jnp.maximum(d0, 1.0)
        pltpu.sync_copy(dB, degc)
        for m in range(BPT // 16):
            d0 = plsc.load_gather(degc, [ixB[pl.ds(16 * m, 16)]])
            rvc[pl.ds(16 * m, 16)] = 1.0 / jnp.maximum(d0, 1.0)
        cpA0.wait()
        cpAt.wait()
        cpB0.wait()
        cpBt.wait()
        outs = [
            pltpu.async_copy(r0a, agOA.at[pl.ds(base, BPT)], sem_o),
            pltpu.async_copy(tba, tOA.at[pl.ds(base, BPT)], sem_o),
            pltpu.async_copy(rva, rvOA.at[pl.ds(base, BPT)], sem_o),
            pltpu.async_copy(r0c, agOB.at[pl.ds(base, BPT)], sem_o),
            pltpu.async_copy(tbc, tOB.at[pl.ds(base, BPT)], sem_o),
            pltpu.async_copy(rvc, rvOB.at[pl.ds(base, BPT)], sem_o),
        ]
        for x in outs:
            x.wait()


_sc_gather = functools.partial(
    pl.kernel,
    out_type=[jax.ShapeDtypeStruct((B, D), _f32),
              jax.ShapeDtypeStruct((B, D), _f32),
              jax.ShapeDtypeStruct((B,), _f32)] * 4,
    mesh=_MESH,
    compiler_params=_SC_PARAMS,
    scratch_types=[
        pltpu.VMEM((BPT,), _i32),
        pltpu.VMEM((BPT,), _i32),
        pltpu.VMEM((BPT, D), _f32),
        pltpu.VMEM((BPT, D), _f32),
        pltpu.VMEM((BPT, D), _f32),
        pltpu.VMEM((BPT, D), _f32),
        pltpu.VMEM((NPAD,), _f32),
        pltpu.VMEM((NPAD,), _f32),
        pltpu.VMEM((BPT,), _f32),
        pltpu.VMEM((BPT,), _f32),
        pltpu.SemaphoreType.DMA,
        pltpu.SemaphoreType.DMA,
    ],
)(_sc_gather_body)


def _tc_reduce_body(uh_ref, ih_ref, cwuv_ref, cwvu_ref, acc_ref):
    i = pl.program_id(0)

    @pl.when(i == 0)
    def _():
        acc_ref[...] = jnp.zeros_like(acc_ref)

    uhb = uh_ref[...]
    ihb = ih_ref[...]
    cu = cwuv_ref[...]
    cv = cwvu_ref[...]
    acc_ref[0:1, :] += jnp.sum(uhb, axis=0, keepdims=True)
    acc_ref[1:2, :] += jnp.sum(ihb, axis=0, keepdims=True)
    acc_ref[2:3, :] += jnp.sum(ihb * cu, axis=0, keepdims=True)
    acc_ref[3:4, :] += jnp.sum(uhb * cv, axis=0, keepdims=True)


def _tc_reduce(uh, ih, cw_uv, cw_vu):
    blk = 1000
    return pl.pallas_call(
        _tc_reduce_body,
        grid=(N // blk,),
        in_specs=[
            pl.BlockSpec((blk, D), lambda i: (i, 0)),
            pl.BlockSpec((blk, D), lambda i: (i, 0)),
            pl.BlockSpec((blk, 1), lambda i: (i, 0)),
            pl.BlockSpec((blk, 1), lambda i: (i, 0)),
        ],
        out_specs=pl.BlockSpec((8, D), lambda i: (0, 0)),
        out_shape=jax.ShapeDtypeStruct((8, D), _f32),
    )(uh, ih, cw_uv, cw_vu)


def _tc_final_body(ag_u, t_u, rv_u, ag_i, t_i, rv_i,
                   ag_fu, t_fu, rv_fu, ag_fi, t_fi, rv_fi, accr,
                   Wu, Wv, l1W, l1b, l2W, l2b, lW, lb, lsW, lsb, fkW, fkb,
                   pr_r, pr_f, lab_r, lab_f):
    inv_n = 1.0 / N
    acc = accr[...]
    S_u = acc[0:1, :] * inv_n
    S_i = acc[1:2, :] * inv_n
    G_icu = acc[2:3, :] * inv_n
    G_uci = acc[3:4, :] * inv_n
    g = jnp.dot(jnp.concatenate([S_u, G_icu], axis=1), l1W[...],
                preferred_element_type=_f32) + l1b[...]
    h = jnp.dot(jnp.concatenate([S_i, G_uci], axis=1), l2W[...],
                preferred_element_type=_f32) + l2b[...]
    st = jax.nn.sigmoid(jnp.dot((g + h) * 0.5, lW[...],
                                preferred_element_type=_f32) + lb[...])
    v = lax.dot_general(st, fkW[...], (((1,), (1,)), ((), ())),
                        preferred_element_type=_f32)

    ru = jnp.maximum(jnp.dot(ag_u[...] * rv_u[...], Wu[...],
                             preferred_element_type=_f32) + t_u[...], 0.0)
    ri = jnp.maximum(jnp.dot(ag_i[...] * rv_i[...], Wv[...],
                             preferred_element_type=_f32) + t_i[...], 0.0)
    fu = jnp.maximum(jnp.dot(ag_fu[...] * rv_fu[...], Wu[...],
                             preferred_element_type=_f32) + t_fu[...], 0.0)
    fi = jnp.maximum(jnp.dot(ag_fi[...] * rv_fi[...], Wv[...],
                             preferred_element_type=_f32) + t_fi[...], 0.0)
    lsW1 = lsW[0:D, :]
    lsW2 = lsW[D:2 * D, :]
    rs = jax.nn.sigmoid(jnp.dot(ru, lsW1, preferred_element_type=_f32)
                        + jnp.dot(ri, lsW2, preferred_element_type=_f32)
                        + lsb[...])
    fs = jax.nn.sigmoid(jnp.dot(fu, lsW1, preferred_element_type=_f32)
                        + jnp.dot(fi, lsW2, preferred_element_type=_f32)
                        + lsb[...])
    bias = fkb[...]
    pr_r[...] = jax.nn.sigmoid(
        jnp.sum(rs * v, axis=1, keepdims=True) + bias)
    pr_f[...] = jax.nn.sigmoid(
        jnp.sum(fs * v, axis=1, keepdims=True) + bias)
    lab_r[...] = jnp.ones_like(lab_r)
    lab_f[...] = jnp.zeros_like(lab_f)


def _tc_final(ag_u, t_u, rv_u, ag_i, t_i, rv_i,
              ag_fu, t_fu, rv_fu, ag_fi, t_fi, rv_fi, acc,
              Wu, Wv, l1W, l1b, l2W, l2b, lW, lb, lsW, lsb, fkW, fkb):
    blk = 512
    row = lambda i: (i, 0)
    fixed = lambda i: (0, 0)
    bspec = pl.BlockSpec((blk, D), row)
    rvspec = pl.BlockSpec((blk, 1), row)
    return pl.pallas_call(
        _tc_final_body,
        grid=(B // blk,),
        in_specs=[bspec, bspec, rvspec] * 4 + [
            pl.BlockSpec((8, D), fixed),
            pl.BlockSpec((D, D), fixed),
            pl.BlockSpec((D, D), fixed),
            pl.BlockSpec((2 * D, D), fixed),
            pl.BlockSpec((1, D), fixed),
            pl.BlockSpec((2 * D, D), fixed),
            pl.BlockSpec((1, D), fixed),
            pl.BlockSpec((D, D), fixed),
            pl.BlockSpec((1, D), fixed),
            pl.BlockSpec((2 * D, D), fixed),
            pl.BlockSpec((1, D), fixed),
            pl.BlockSpec((D, D), fixed),
            pl.BlockSpec((1, 1), fixed),
        ],
        out_specs=[pl.BlockSpec((blk, 1), row)] * 4,
        out_shape=[jax.ShapeDtypeStruct((B, 1), _f32)] * 4,
    )(ag_u, t_u, rv_u, ag_i, t_i, rv_i,
      ag_fu, t_fu, rv_fu, ag_fi, t_fi, rv_fi, acc,
      Wu, Wv, l1W, l1b, l2W, l2b, lW, lb, lsW, lsb, fkW, fkb)


def _2d(a):
    return a.reshape(ER, CH)


def _3d(a):
    return a.reshape(2, ER, CH)


def kernel(user_hidden_out, item_hidden_out, fake_user_hidden_out,
           fake_item_hidden_out, UV_adj, VU_adj, CUV_adj, CVU_adj,
           user_One, item_One, UV_rated, UV_rated_vals, VU_rated,
           VU_rated_vals, relation_UV_adj, lin1_W, lin1_b, lin2_W, lin2_b,
           lin_W, lin_b, lin_sub_W, lin_sub_b, fk_W, fk_b, gat_Wu, gat_Wv):
    del relation_UV_adj
    zrows = jnp.zeros((640, D), _f32)
    z640 = jnp.zeros((640,), _f32)

    (agg_u, agg_i, agg_fu, agg_fi,
     deg_u, deg_i, deg_fu, deg_fi) = _sc_agg(
        user_hidden_out, item_hidden_out, fake_user_hidden_out,
        fake_item_hidden_out,
        _3d(UV_adj), _3d(VU_adj), _3d(CUV_adj), _3d(CVU_adj),
        zrows, z640)

    cw_uv, cw_vu = _sc_cw(
        _3d(UV_rated), _2d(UV_rated_vals),
        _3d(VU_rated), _2d(VU_rated_vals), z640)

    (ag_u, t_u, rv_u, ag_i, t_i, rv_i, ag_fu, t_fu, rv_fu,
     ag_fi, t_fi, rv_fi) = _sc_gather(
        agg_u, agg_i, agg_fu, agg_fi,
        deg_u, deg_i, deg_fu, deg_fi,
        user_hidden_out, item_hidden_out, fake_user_hidden_out,
        fake_item_hidden_out, user_One, item_One)

    acc = _tc_reduce(user_hidden_out, item_hidden_out,
                     cw_uv[:N, None], cw_vu[:N, None])

    pr_r, pr_f, lab_r, lab_f = _tc_final(
        ag_u, t_u, rv_u[:, None], ag_i, t_i, rv_i[:, None],
        ag_fu, t_fu, rv_fu[:, None], ag_fi, t_fi, rv_fi[:, None], acc,
        gat_Wu, gat_Wv, lin1_W, lin1_b.reshape(1, D), lin2_W,
        lin2_b.reshape(1, D), lin_W, lin_b.reshape(1, D), lin_sub_W,
        lin_sub_b.reshape(1, D), fk_W, fk_b.reshape(1, 1))

    prob = jnp.concatenate([pr_r.reshape(B), pr_f.reshape(B)])
    label = jnp.concatenate([lab_r.reshape(B), lab_f.reshape(B)])
    return prob, label
